# baseline (speedup 1.0000x reference)
import functools

import jax
import jax.numpy as jnp
from jax import lax
from jax.experimental import pallas as pl
from jax.experimental.pallas import tpu as pltpu
from jax.experimental.pallas import tpu_sc as plsc

NC = 2
NS = 16
NW = NC * NS
C = 128


def _pack_bf16_halves(x):
    Hh = x.shape[-1] // 2
    u = lax.bitcast_convert_type(x, jnp.uint32)
    r = u + jnp.uint32(0x7FFF) + ((u >> 16) & jnp.uint32(1))
    lo = r[:, :Hh] >> 16
    hi = r[:, Hh:] & jnp.uint32(0xFFFF0000)
    return lax.bitcast_convert_type(lo | hi, jnp.int32)


def _unpack_bf16_halves(p):
    u = lax.bitcast_convert_type(p, jnp.uint32)
    lo = lax.bitcast_convert_type(u << 16, jnp.float32)
    hi = lax.bitcast_convert_type(u & jnp.uint32(0xFFFF0000), jnp.float32)
    return jnp.concatenate([lo, hi], axis=-1)


def _tc_node_pre(f, W1a, W1b, b1r):
    N, H = f.shape
    BN = 1000
    grid = (N // BN,)

    def body(f_ref, a_ref, b_ref, bias_ref, fa_ref, fb_ref):
        fblk = f_ref[...]
        fa_ref[...] = jnp.dot(fblk, a_ref[...],
                              preferred_element_type=jnp.float32) + bias_ref[...]
        fb_ref[...] = jnp.dot(fblk, b_ref[...],
                              preferred_element_type=jnp.float32)

    return pl.pallas_call(
        body,
        grid=grid,
        in_specs=[
            pl.BlockSpec((BN, H), lambda i: (i, 0)),
            pl.BlockSpec((H, H), lambda i: (0, 0)),
            pl.BlockSpec((H, H), lambda i: (0, 0)),
            pl.BlockSpec((1, H), lambda i: (0, 0)),
        ],
        out_specs=[
            pl.BlockSpec((BN, H), lambda i: (i, 0)),
            pl.BlockSpec((BN, H), lambda i: (i, 0)),
        ],
        out_shape=[
            jax.ShapeDtypeStruct((N, H), jnp.float32),
            jax.ShapeDtypeStruct((N, H), jnp.float32),
        ],
    )(f, W1a, W1b, b1r)


def _tc_message(pre32, w, w1cq, W2q, b2r):
    E, H2 = pre32.shape
    H = H2 * 2
    BE = next(b for b in (3200, 2560, 2000, 1600, 1000, 800) if E % b == 0)
    grid = (E // BE,)

    def body(pre_ref, w_ref, w1c_ref, w2_ref, b2_ref, m_ref):
        wv = w_ref[...]
        x = _unpack_bf16_halves(pre_ref[...]) + (wv * wv) * w1c_ref[...]
        h = jnp.maximum(x, 0.0).astype(jnp.bfloat16)
        z = jnp.dot(h, w2_ref[...],
                    preferred_element_type=jnp.float32) + b2_ref[...]
        msg = jnp.maximum(z, 0.0)
        m_ref[...] = msg * jax.nn.sigmoid(msg)

    return pl.pallas_call(
        body,
        grid=grid,
        in_specs=[
            pl.BlockSpec((BE, H2), lambda i: (i, 0)),
            pl.BlockSpec((BE, 1), lambda i: (i, 0)),
            pl.BlockSpec((1, H), lambda i: (0, 0)),
            pl.BlockSpec((H, H), lambda i: (0, 0)),
            pl.BlockSpec((1, H), lambda i: (0, 0)),
        ],
        out_specs=pl.BlockSpec((BE, H), lambda i: (i, 0)),
        out_shape=jax.ShapeDtypeStruct((E, H), jnp.float32),
    )(pre32, w, w1cq, W2q, b2r)


def _tc_update(parts, f, U1, ub1r, U2, ub2r):
    N, H = f.shape
    BN = 1000
    grid = (N // BN,)
    np_ = len(parts)

    def body(*refs):
        part_refs = refs[:np_]
        f_ref, u1_ref, ub1_ref, u2_ref, ub2_ref, o_ref = refs[np_:]
        fblk = f_ref[...]
        inp = fblk
        for pr in part_refs:
            inp = inp + pr[...]
        h1 = jnp.maximum(
            jnp.dot(inp, u1_ref[...],
                    preferred_element_type=jnp.float32) + ub1_ref[...], 0.0)
        o_ref[...] = jnp.dot(h1, u2_ref[...],
                             preferred_element_type=jnp.float32) + ub2_ref[...] + fblk

    return pl.pallas_call(
        body,
        grid=grid,
        in_specs=[pl.BlockSpec((BN, H), lambda i: (i, 0)) for _ in range(np_)]
        + [
            pl.BlockSpec((BN, H), lambda i: (i, 0)),
            pl.BlockSpec((H, H), lambda i: (0, 0)),
            pl.BlockSpec((1, H), lambda i: (0, 0)),
            pl.BlockSpec((H, H), lambda i: (0, 0)),
            pl.BlockSpec((1, H), lambda i: (0, 0)),
        ],
        out_specs=pl.BlockSpec((BN, H), lambda i: (i, 0)),
        out_shape=jax.ShapeDtypeStruct((N, H), jnp.float32),
    )(*parts, f, U1, ub1r, U2, ub2r)


def _sc_gather(fA, fB, src, dst):
    N, H = fA.shape
    E = src.shape[0]
    nchunk = E // C
    mesh = plsc.VectorSubcoreMesh(core_axis_name="c", subcore_axis_name="s")

    @functools.partial(
        pl.kernel,
        out_type=jax.ShapeDtypeStruct((E, H // 2), jnp.int32),
        mesh=mesh,
        compiler_params=pltpu.CompilerParams(needs_layout_passes=False),
        scratch_types=[
            pltpu.VMEM((2, C), jnp.int32),
            pltpu.VMEM((2, C), jnp.int32),
            pltpu.VMEM((2, C, H), jnp.float32),
            pltpu.VMEM((2, C, H), jnp.float32),
            pltpu.VMEM((2, C, H // 2), jnp.int32),
            pltpu.SemaphoreType.DMA((2,)),
            pltpu.SemaphoreType.DMA((2,)),
            pltpu.SemaphoreType.DMA((2,)),
            pltpu.SemaphoreType.DMA((2,)),
            pltpu.SemaphoreType.DMA((2,)),
        ],
    )
    def run(fa_hbm, fb_hbm, src_hbm, dst_hbm, pre_hbm,
            idx_a, idx_b, buf_a, buf_b, buf_o, sia, sib, sga, sgb, sst):
        cid = lax.axis_index("c")
        sid = lax.axis_index("s")
        wid = sid * NC + cid
        n = (nchunk // NW) + jnp.where(wid < (nchunk % NW), 1, 0)

        def cbase(t):
            return (wid + t * NW) * C

        def issue_idx(t, q):
            b = cbase(t)
            pltpu.async_copy(src_hbm.at[pl.ds(b, C)], idx_a.at[q], sia.at[q])
            pltpu.async_copy(dst_hbm.at[pl.ds(b, C)], idx_b.at[q], sib.at[q])

        def wait_idx(q):
            pltpu.make_async_copy(src_hbm.at[pl.ds(0, C)], idx_a.at[q], sia.at[q]).wait()
            pltpu.make_async_copy(dst_hbm.at[pl.ds(0, C)], idx_b.at[q], sib.at[q]).wait()

        def issue_gathers(q):
            pltpu.async_copy(fa_hbm.at[idx_a.at[q]], buf_a.at[q], sga.at[q])
            pltpu.async_copy(fb_hbm.at[idx_b.at[q]], buf_b.at[q], sgb.at[q])

        def wait_gathers(q):
            pltpu.make_async_copy(fa_hbm.at[idx_a.at[q]], buf_a.at[q], sga.at[q]).wait()
            pltpu.make_async_copy(fb_hbm.at[idx_b.at[q]], buf_b.at[q], sgb.at[q]).wait()

        def issue_store(t, q):
            pltpu.async_copy(buf_o.at[q], pre_hbm.at[pl.ds(cbase(t), C)], sst.at[q])

        def wait_store(q):
            pltpu.make_async_copy(buf_o.at[q], pre_hbm.at[pl.ds(0, C)], sst.at[q]).wait()

        def vadd(q):
            def vrow(i, c2):
                for j in range(H // 32):
                    lo = (buf_a[q, i, pl.ds(j * 32, 16)]
                          + buf_b[q, i, pl.ds(j * 32, 16)])
                    hi = (buf_a[q, i, pl.ds(j * 32 + 16, 16)]
                          + buf_b[q, i, pl.ds(j * 32 + 16, 16)])
                    p = plsc.pack(lo, hi, format=plsc.PackFormat.INTERLEAVED)
                    buf_o[q, i, pl.ds(j * 16, 16)] = plsc.bitcast(p, jnp.int32)
                return c2

            lax.fori_loop(0, C, vrow, 0)

        def halfstep(t, q):
            wait_gathers(q)
            vadd(q)
            issue_store(t, q)

            @pl.when(t + 1 < n)
            def _():
                @pl.when(t >= 1)
                def _():
                    wait_store(1 - q)
                wait_idx(1 - q)
                issue_gathers(1 - q)

            @pl.when(t + 2 < n)
            def _():
                issue_idx(t + 2, q)

        issue_idx(0, 0)
        wait_idx(0)
        issue_gathers(0)

        @pl.when(n > 1)
        def _():
            issue_idx(1, 1)

        def body(g, carry):
            t0 = 2 * g
            halfstep(t0, 0)

            @pl.when(t0 + 1 < n)
            def _():
                halfstep(t0 + 1, 1)

            return carry

        lax.fori_loop(0, lax.div(n + 1, 2), body, 0)

        @pl.when(lax.rem(n, 2) == 1)
        def _():
            @pl.when(n > 1)
            def _():
                wait_store(1)
            wait_store(0)

        @pl.when(lax.rem(n, 2) == 0)
        def _():
            @pl.when(n > 1)
            def _():
                wait_store(0)
            wait_store(1)

    return run(fA, fB, src, dst)


def _sc_scatter(m, dst, N):
    E, H = m.shape
    nchunk = E // C
    half = nchunk // NC
    ZB = 80
    nblk = N // ZB
    mesh = plsc.VectorSubcoreMesh(core_axis_name="c", subcore_axis_name="s")

    @functools.partial(
        pl.kernel,
        out_type=[
            jax.ShapeDtypeStruct((N, H), jnp.float32),
            jax.ShapeDtypeStruct((N, H), jnp.float32),
        ],
        mesh=mesh,
        scratch_types=[
            pltpu.VMEM((2, C), jnp.int32),
            pltpu.VMEM((2, C, H), jnp.float32),
            pltpu.VMEM((ZB, H), jnp.float32),
            pltpu.VMEM_SHARED((N, H), jnp.float32),
            pltpu.SemaphoreType.DMA((2,)),
            pltpu.SemaphoreType.DMA((2,)),
            pltpu.SemaphoreType.DMA((2,)),
        ],
    )
    def run(m_hbm, dst_hbm, out0_hbm, out1_hbm,
            idx_v, buf_m, buf_z, acc, sii, sim, ssc):
        cid = lax.axis_index("c")
        sid = lax.axis_index("s")

        def zrow(i, carry):
            def zcol(j, carry2):
                buf_z[i, pl.ds(j * 16, 16)] = jnp.zeros((16,), jnp.float32)
                return carry2
            return lax.fori_loop(0, H // 16, zcol, carry)

        lax.fori_loop(0, ZB, zrow, 0)
        nb = (nblk // NS) + jnp.where(sid < (nblk % NS), 1, 0)

        def zblk(t, carry):
            pltpu.sync_copy(buf_z, acc.at[pl.ds((sid + t * NS) * ZB, ZB)])
            return carry

        lax.fori_loop(0, nb, zblk, 0)
        plsc.subcore_barrier()

        n = (half // NS) + jnp.where(sid < (half % NS), 1, 0)

        def cbase(t):
            return (cid * half + sid + t * NS) * C

        def issue_loads(t, q):
            b = cbase(t)
            pltpu.async_copy(dst_hbm.at[pl.ds(b, C)], idx_v.at[q], sii.at[q])
            pltpu.async_copy(m_hbm.at[pl.ds(b, C)], buf_m.at[q], sim.at[q])

        def wait_loads(q):
            pltpu.make_async_copy(dst_hbm.at[pl.ds(0, C)], idx_v.at[q], sii.at[q]).wait()
            pltpu.make_async_copy(m_hbm.at[pl.ds(0, C)], buf_m.at[q], sim.at[q]).wait()

        def issue_scatter(q):
            pltpu.async_copy(buf_m.at[q], acc.at[idx_v.at[q]], ssc.at[q], add=True)

        def wait_scatter(q):
            pltpu.make_async_copy(buf_m.at[q], acc.at[idx_v.at[q]], ssc.at[q]).wait()

        def halfstep(t, q):
            wait_loads(q)
            issue_scatter(q)

            @pl.when(t + 2 < n)
            def _():
                wait_scatter(q)
                issue_loads(t + 2, q)

        @pl.when(n > 0)
        def _():
            issue_loads(0, 0)

        @pl.when(n > 1)
        def _():
            issue_loads(1, 1)

        def body(g, carry):
            t0 = 2 * g

            @pl.when(t0 < n)
            def _():
                halfstep(t0, 0)

            @pl.when(t0 + 1 < n)
            def _():
                halfstep(t0 + 1, 1)

            return carry

        lax.fori_loop(0, lax.div(n + 1, 2), body, 0)

        @pl.when(n > 1)
        def _():
            wait_scatter(lax.rem(n, 2))

        @pl.when(n > 0)
        def _():
            wait_scatter(lax.rem(n + 1, 2))

        plsc.subcore_barrier()

        def wblk(t, carry):
            sl = pl.ds((sid + t * NS) * ZB, ZB)

            @pl.when(cid == 0)
            def _copy0():
                pltpu.sync_copy(acc.at[sl], out0_hbm.at[sl])

            @pl.when(cid == 1)
            def _copy1():
                pltpu.sync_copy(acc.at[sl], out1_hbm.at[sl])

            return carry

        lax.fori_loop(0, nb, wblk, 0)

    return run(m, dst)


def kernel(f, edge_index, w, W1, b1, W2, b2, U1, ub1, U2, ub2):
    N, H = f.shape
    src = edge_index[0]
    dst = edge_index[1]
    W1a = W1[0:H]
    W1b = W1[H:2 * H]
    w1c = W1[2 * H:2 * H + 1]

    fA, fB = _tc_node_pre(f, W1a, W1b, b1.reshape(1, H))

    q_perm = jnp.concatenate(
        [jnp.arange(32 * j, 32 * j + 16) for j in range(H // 32)]
        + [jnp.arange(32 * j + 16, 32 * j + 32) for j in range(H // 32)])
    w1cq = w1c[:, q_perm]
    W2q = W2[q_perm, :].astype(jnp.bfloat16)

    E = src.shape[0]
    Eh = E // 2
    b2r = b2.reshape(1, H)
    parts = []
    for h in range(2):
        sl = slice(h * Eh, (h + 1) * Eh)
        pre_h = _sc_gather(fA, fB, src[sl], dst[sl])
        m_h = _tc_message(pre_h, w[sl], w1cq, W2q, b2r)
        p0, p1 = _sc_scatter(m_h, dst[sl], N)
        parts += [p0, p1]
    return _tc_update(parts, f, U1, ub1.reshape(1, H), U2, ub2.reshape(1, H))


# speedup vs baseline: 1.0495x; 1.0495x over previous
import functools

import jax
import jax.numpy as jnp
from jax import lax
from jax.experimental import pallas as pl
from jax.experimental.pallas import tpu as pltpu
from jax.experimental.pallas import tpu_sc as plsc

NC = 2
NS = 16
NW = NC * NS
C = 128


def _pack_bf16_halves(x):
    Hh = x.shape[-1] // 2
    u = lax.bitcast_convert_type(x, jnp.uint32)
    r = u + jnp.uint32(0x7FFF) + ((u >> 16) & jnp.uint32(1))
    lo = r[:, :Hh] >> 16
    hi = r[:, Hh:] & jnp.uint32(0xFFFF0000)
    return lax.bitcast_convert_type(lo | hi, jnp.int32)


def _unpack_bf16_halves(p):
    u = lax.bitcast_convert_type(p, jnp.uint32)
    lo = lax.bitcast_convert_type(u << 16, jnp.float32)
    hi = lax.bitcast_convert_type(u & jnp.uint32(0xFFFF0000), jnp.float32)
    return jnp.concatenate([lo, hi], axis=-1)


def _tc_node_pre(f, W1a, W1b, b1r):
    N, H = f.shape
    BN = 1000
    grid = (N // BN,)

    def body(f_ref, a_ref, b_ref, bias_ref, fa_ref, fb_ref):
        fblk = f_ref[...]
        fa_ref[...] = jnp.dot(fblk, a_ref[...],
                              preferred_element_type=jnp.float32) + bias_ref[...]
        fb_ref[...] = jnp.dot(fblk, b_ref[...],
                              preferred_element_type=jnp.float32)

    return pl.pallas_call(
        body,
        grid=grid,
        in_specs=[
            pl.BlockSpec((BN, H), lambda i: (i, 0)),
            pl.BlockSpec((H, H), lambda i: (0, 0)),
            pl.BlockSpec((H, H), lambda i: (0, 0)),
            pl.BlockSpec((1, H), lambda i: (0, 0)),
        ],
        out_specs=[
            pl.BlockSpec((BN, H), lambda i: (i, 0)),
            pl.BlockSpec((BN, H), lambda i: (i, 0)),
        ],
        out_shape=[
            jax.ShapeDtypeStruct((N, H), jnp.float32),
            jax.ShapeDtypeStruct((N, H), jnp.float32),
        ],
    )(f, W1a, W1b, b1r)


def _tc_message(pre, w, w1c, W2, b2r):
    E, H = pre.shape
    BE = next(b for b in (3200, 2560, 2000, 1600, 1000, 800) if E % b == 0)
    grid = (E // BE,)

    def body(pre_ref, w_ref, w1c_ref, w2_ref, b2_ref, m_ref):
        wv = w_ref[...]
        x = pre_ref[...] + (wv * wv) * w1c_ref[...]
        h = jnp.maximum(x, 0.0).astype(jnp.bfloat16)
        z = jnp.dot(h, w2_ref[...],
                    preferred_element_type=jnp.float32) + b2_ref[...]
        msg = jnp.maximum(z, 0.0)
        m_ref[...] = msg * jax.nn.sigmoid(msg)

    return pl.pallas_call(
        body,
        grid=grid,
        in_specs=[
            pl.BlockSpec((BE, H), lambda i: (i, 0)),
            pl.BlockSpec((BE, 1), lambda i: (i, 0)),
            pl.BlockSpec((1, H), lambda i: (0, 0)),
            pl.BlockSpec((H, H), lambda i: (0, 0)),
            pl.BlockSpec((1, H), lambda i: (0, 0)),
        ],
        out_specs=pl.BlockSpec((BE, H), lambda i: (i, 0)),
        out_shape=jax.ShapeDtypeStruct((E, H), jnp.float32),
    )(pre, w, w1c, W2, b2r)


def _tc_update(parts, f, U1, ub1r, U2, ub2r):
    N, H = f.shape
    BN = 1000
    grid = (N // BN,)
    np_ = len(parts)

    def body(*refs):
        part_refs = refs[:np_]
        f_ref, u1_ref, ub1_ref, u2_ref, ub2_ref, o_ref = refs[np_:]
        fblk = f_ref[...]
        inp = fblk
        for pr in part_refs:
            inp = inp + pr[...]
        h1 = jnp.maximum(
            jnp.dot(inp, u1_ref[...],
                    preferred_element_type=jnp.float32) + ub1_ref[...], 0.0)
        o_ref[...] = jnp.dot(h1, u2_ref[...],
                             preferred_element_type=jnp.float32) + ub2_ref[...] + fblk

    return pl.pallas_call(
        body,
        grid=grid,
        in_specs=[pl.BlockSpec((BN, H), lambda i: (i, 0)) for _ in range(np_)]
        + [
            pl.BlockSpec((BN, H), lambda i: (i, 0)),
            pl.BlockSpec((H, H), lambda i: (0, 0)),
            pl.BlockSpec((1, H), lambda i: (0, 0)),
            pl.BlockSpec((H, H), lambda i: (0, 0)),
            pl.BlockSpec((1, H), lambda i: (0, 0)),
        ],
        out_specs=pl.BlockSpec((BN, H), lambda i: (i, 0)),
        out_shape=jax.ShapeDtypeStruct((N, H), jnp.float32),
    )(*parts, f, U1, ub1r, U2, ub2r)


def _sc_gather(fA, fB, src, dst):
    N, H = fA.shape
    E = src.shape[0]
    nchunk = E // C
    mesh = plsc.VectorSubcoreMesh(core_axis_name="c", subcore_axis_name="s")

    @functools.partial(
        pl.kernel,
        out_type=jax.ShapeDtypeStruct((E, H), jnp.float32),
        mesh=mesh,
        scratch_types=[
            pltpu.VMEM((2, C), jnp.int32),
            pltpu.VMEM((2, C), jnp.int32),
            pltpu.VMEM((2, C, H), jnp.float32),
            pltpu.VMEM((2, C, H), jnp.float32),
            pltpu.SemaphoreType.DMA((2,)),
            pltpu.SemaphoreType.DMA((2,)),
            pltpu.SemaphoreType.DMA((2,)),
            pltpu.SemaphoreType.DMA((2,)),
            pltpu.SemaphoreType.DMA((2,)),
        ],
    )
    def run(fa_hbm, fb_hbm, src_hbm, dst_hbm, pre_hbm,
            idx_a, idx_b, buf_a, buf_b, sia, sib, sga, sgb, sst):
        cid = lax.axis_index("c")
        sid = lax.axis_index("s")
        wid = sid * NC + cid
        n = (nchunk // NW) + jnp.where(wid < (nchunk % NW), 1, 0)

        def cbase(t):
            return (wid + t * NW) * C

        def issue_idx(t, q):
            b = cbase(t)
            pltpu.async_copy(src_hbm.at[pl.ds(b, C)], idx_a.at[q], sia.at[q])
            pltpu.async_copy(dst_hbm.at[pl.ds(b, C)], idx_b.at[q], sib.at[q])

        def wait_idx(q):
            pltpu.make_async_copy(src_hbm.at[pl.ds(0, C)], idx_a.at[q], sia.at[q]).wait()
            pltpu.make_async_copy(dst_hbm.at[pl.ds(0, C)], idx_b.at[q], sib.at[q]).wait()

        def issue_gathers(q):
            pltpu.async_copy(fa_hbm.at[idx_a.at[q]], buf_a.at[q], sga.at[q])
            pltpu.async_copy(fb_hbm.at[idx_b.at[q]], buf_b.at[q], sgb.at[q])

        def wait_gathers(q):
            pltpu.make_async_copy(fa_hbm.at[idx_a.at[q]], buf_a.at[q], sga.at[q]).wait()
            pltpu.make_async_copy(fb_hbm.at[idx_b.at[q]], buf_b.at[q], sgb.at[q]).wait()

        def issue_store(t, q):
            pltpu.async_copy(buf_a.at[q], pre_hbm.at[pl.ds(cbase(t), C)], sst.at[q])

        def wait_store(q):
            pltpu.make_async_copy(buf_a.at[q], pre_hbm.at[pl.ds(0, C)], sst.at[q]).wait()

        def vadd(q):
            def vrow(i, c2):
                for j in range(H // 16):
                    sl = (q, i, pl.ds(j * 16, 16))
                    buf_a[sl] = buf_a[sl] + buf_b[sl]
                return c2

            lax.fori_loop(0, C, vrow, 0)

        def halfstep(t, q):
            wait_gathers(q)

            @pl.when(t + 1 < n)
            def _():
                @pl.when(t >= 1)
                def _():
                    wait_store(1 - q)
                wait_idx(1 - q)
                issue_gathers(1 - q)

            vadd(q)
            issue_store(t, q)

            @pl.when(t + 2 < n)
            def _():
                issue_idx(t + 2, q)

        issue_idx(0, 0)
        wait_idx(0)
        issue_gathers(0)

        @pl.when(n > 1)
        def _():
            issue_idx(1, 1)

        def body(g, carry):
            t0 = 2 * g
            halfstep(t0, 0)

            @pl.when(t0 + 1 < n)
            def _():
                halfstep(t0 + 1, 1)

            return carry

        lax.fori_loop(0, lax.div(n + 1, 2), body, 0)

        @pl.when(lax.rem(n, 2) == 1)
        def _():
            @pl.when(n > 1)
            def _():
                wait_store(1)
            wait_store(0)

        @pl.when(lax.rem(n, 2) == 0)
        def _():
            @pl.when(n > 1)
            def _():
                wait_store(0)
            wait_store(1)

    return run(fA, fB, src, dst)


def _sc_scatter(m, dst, N):
    E, H = m.shape
    nchunk = E // C
    half = nchunk // NC
    ZB = 80
    nblk = N // ZB
    mesh = plsc.VectorSubcoreMesh(core_axis_name="c", subcore_axis_name="s")

    @functools.partial(
        pl.kernel,
        out_type=[
            jax.ShapeDtypeStruct((N, H), jnp.float32),
            jax.ShapeDtypeStruct((N, H), jnp.float32),
        ],
        mesh=mesh,
        scratch_types=[
            pltpu.VMEM((2, C), jnp.int32),
            pltpu.VMEM((2, C, H), jnp.float32),
            pltpu.VMEM((ZB, H), jnp.float32),
            pltpu.VMEM_SHARED((N, H), jnp.float32),
            pltpu.SemaphoreType.DMA((2,)),
            pltpu.SemaphoreType.DMA((2,)),
            pltpu.SemaphoreType.DMA((2,)),
        ],
    )
    def run(m_hbm, dst_hbm, out0_hbm, out1_hbm,
            idx_v, buf_m, buf_z, acc, sii, sim, ssc):
        cid = lax.axis_index("c")
        sid = lax.axis_index("s")

        def zrow(i, carry):
            def zcol(j, carry2):
                buf_z[i, pl.ds(j * 16, 16)] = jnp.zeros((16,), jnp.float32)
                return carry2
            return lax.fori_loop(0, H // 16, zcol, carry)

        lax.fori_loop(0, ZB, zrow, 0)
        nb = (nblk // NS) + jnp.where(sid < (nblk % NS), 1, 0)

        def zblk(t, carry):
            pltpu.sync_copy(buf_z, acc.at[pl.ds((sid + t * NS) * ZB, ZB)])
            return carry

        lax.fori_loop(0, nb, zblk, 0)
        plsc.subcore_barrier()

        n = (half // NS) + jnp.where(sid < (half % NS), 1, 0)

        def cbase(t):
            return (cid * half + sid + t * NS) * C

        def issue_loads(t, q):
            b = cbase(t)
            pltpu.async_copy(dst_hbm.at[pl.ds(b, C)], idx_v.at[q], sii.at[q])
            pltpu.async_copy(m_hbm.at[pl.ds(b, C)], buf_m.at[q], sim.at[q])

        def wait_loads(q):
            pltpu.make_async_copy(dst_hbm.at[pl.ds(0, C)], idx_v.at[q], sii.at[q]).wait()
            pltpu.make_async_copy(m_hbm.at[pl.ds(0, C)], buf_m.at[q], sim.at[q]).wait()

        def issue_scatter(q):
            pltpu.async_copy(buf_m.at[q], acc.at[idx_v.at[q]], ssc.at[q], add=True)

        def wait_scatter(q):
            pltpu.make_async_copy(buf_m.at[q], acc.at[idx_v.at[q]], ssc.at[q]).wait()

        def halfstep(t, q):
            wait_loads(q)
            issue_scatter(q)

            @pl.when(t + 2 < n)
            def _():
                wait_scatter(q)
                issue_loads(t + 2, q)

        @pl.when(n > 0)
        def _():
            issue_loads(0, 0)

        @pl.when(n > 1)
        def _():
            issue_loads(1, 1)

        def body(g, carry):
            t0 = 2 * g

            @pl.when(t0 < n)
            def _():
                halfstep(t0, 0)

            @pl.when(t0 + 1 < n)
            def _():
                halfstep(t0 + 1, 1)

            return carry

        lax.fori_loop(0, lax.div(n + 1, 2), body, 0)

        @pl.when(n > 1)
        def _():
            wait_scatter(lax.rem(n, 2))

        @pl.when(n > 0)
        def _():
            wait_scatter(lax.rem(n + 1, 2))

        plsc.subcore_barrier()

        def wblk(t, carry):
            sl = pl.ds((sid + t * NS) * ZB, ZB)

            @pl.when(cid == 0)
            def _copy0():
                pltpu.sync_copy(acc.at[sl], out0_hbm.at[sl])

            @pl.when(cid == 1)
            def _copy1():
                pltpu.sync_copy(acc.at[sl], out1_hbm.at[sl])

            return carry

        lax.fori_loop(0, nb, wblk, 0)

    return run(m, dst)


def kernel(f, edge_index, w, W1, b1, W2, b2, U1, ub1, U2, ub2):
    N, H = f.shape
    src = edge_index[0]
    dst = edge_index[1]
    W1a = W1[0:H]
    W1b = W1[H:2 * H]
    w1c = W1[2 * H:2 * H + 1]

    fA, fB = _tc_node_pre(f, W1a, W1b, b1.reshape(1, H))

    E = src.shape[0]
    Eh = E // 2
    W2b = W2.astype(jnp.bfloat16)
    b2r = b2.reshape(1, H)
    parts = []
    for h in range(2):
        sl = slice(h * Eh, (h + 1) * Eh)
        pre_h = _sc_gather(fA, fB, src[sl], dst[sl])
        m_h = _tc_message(pre_h, w[sl], w1c, W2b, b2r)
        p0, p1 = _sc_scatter(m_h, dst[sl], N)
        parts += [p0, p1]
    return _tc_update(parts, f, U1, ub1.reshape(1, H), U2, ub2.reshape(1, H))
